# baseline (speedup 1.0000x reference)
import functools

import jax
import jax.numpy as jnp
from jax.experimental import pallas as pl
from jax.experimental.pallas import tpu as pltpu

N_ATOMS = 4096
N_FEAT = 128
N_BOND = 4
N_OUT = 32
BM = 256


def _molconv_kernel(af_ref, wt_ref, b_ref, bond_ref, out_ref, h_ref):
    @pl.when(pl.program_id(0) == 0)
    def _compute_h():
        h = jnp.dot(af_ref[...], wt_ref[...], preferred_element_type=jnp.float32)
        h_ref[...] = h + b_ref[...]

    bond = bond_ref[...]
    h = h_ref[...]
    acc = jnp.zeros((BM, N_OUT), dtype=jnp.float32)
    for bt in range(N_BOND):
        acc += jnp.dot(
            bond[:, bt * N_ATOMS:(bt + 1) * N_ATOMS],
            h[:, bt * N_OUT:(bt + 1) * N_OUT],
            preferred_element_type=jnp.float32,
        )
    out_ref[...] = acc


@functools.partial(jax.jit, static_argnames=())
def kernel(atom_features, bond_info, W, b):
    n = atom_features.shape[0]
    wt = W.T
    b2 = b.reshape(1, N_BOND * N_OUT)
    grid = (n // BM,)
    return pl.pallas_call(
        _molconv_kernel,
        grid=grid,
        in_specs=[
            pl.BlockSpec((n, N_FEAT), lambda i: (0, 0)),
            pl.BlockSpec((N_FEAT, N_BOND * N_OUT), lambda i: (0, 0)),
            pl.BlockSpec((1, N_BOND * N_OUT), lambda i: (0, 0)),
            pl.BlockSpec((BM, N_BOND * n), lambda i: (i, 0)),
        ],
        out_specs=pl.BlockSpec((BM, N_OUT), lambda i: (i, 0)),
        out_shape=jax.ShapeDtypeStruct((n, N_OUT), jnp.float32),
        scratch_shapes=[pltpu.VMEM((n, N_BOND * N_OUT), jnp.float32)],
    )(atom_features, wt, b2, bond_info)
